# baseline (speedup 1.0000x reference)
import functools

import jax
import jax.numpy as jnp
import numpy as np
from jax import lax
from jax.experimental import pallas as pl
from jax.experimental.pallas import tpu as pltpu
from jax.experimental.pallas import tpu_sc as plsc

_EMBED = 128
_LANES = 16
_NUM_WORKERS = 32
_DEPTH = 4


def _make_pe(maxlen: int, d: int) -> np.ndarray:
    pe = np.zeros((maxlen, d), dtype=np.float32)
    position = np.arange(0, maxlen)[:, np.newaxis]
    div_term = np.exp(np.arange(0, d, 2) * -(np.log(10000.0) / d))
    pe[:, 0::2] = np.sin(position * div_term)
    pe[:, 1::2] = np.cos(position * div_term)
    return pe


@functools.partial(jax.jit, static_argnums=(2, 3))
def _embed_fixed(x_flat, w, b, l):
    d = w.shape[1]
    n = b * l
    rows_per_w = n // _NUM_WORKERS
    seq_per_w = b // _NUM_WORKERS
    chunks = [(0, 128), (128, l - 128)] if l > 128 else [(0, l)]
    nc = len(chunks)
    pe = jnp.asarray(_make_pe(l, d))
    assert seq_per_w % _DEPTH == 0

    mesh = plsc.VectorSubcoreMesh(core_axis_name="c", subcore_axis_name="s")

    @functools.partial(
        pl.kernel,
        out_type=jax.ShapeDtypeStruct((n, d), jnp.float32),
        mesh=mesh,
        scratch_types=[
            pltpu.VMEM((l, d), jnp.float32),
            [pltpu.VMEM((l,), jnp.int32) for _ in range(_DEPTH)],
            [pltpu.VMEM((l, d), jnp.float32) for _ in range(_DEPTH)],
            [pltpu.SemaphoreType.DMA for _ in range(_DEPTH)],
            [[pltpu.SemaphoreType.DMA for _ in chunks] for _ in range(_DEPTH)],
            [pltpu.SemaphoreType.DMA for _ in range(_DEPTH)],
        ],
    )
    def run(x_hbm, pe_hbm, w_hbm, out_hbm,
            pe_v, idxs, bufs, isems, gsems, ssems):
        wid = lax.axis_index("s") * 2 + lax.axis_index("c")
        base = wid * rows_per_w
        pltpu.sync_copy(pe_hbm, pe_v)

        def idx_desc(s, p):
            return pltpu.make_async_copy(
                x_hbm.at[pl.ds(base + s * l, l)], idxs[p], isems[p]
            )

        def gather_desc(p, ci, add=False):
            off, sz = chunks[ci]
            if add:
                pltpu.async_copy(
                    w_hbm.at[idxs[p].at[pl.ds(off, sz)]],
                    bufs[p].at[pl.ds(off, sz)],
                    gsems[p][ci],
                    add=True,
                )
                return None
            return pltpu.make_async_copy(
                w_hbm.at[idxs[p].at[pl.ds(off, sz)]],
                bufs[p].at[pl.ds(off, sz)],
                gsems[p][ci],
            )

        def store_desc(s, p):
            return pltpu.make_async_copy(
                bufs[p], out_hbm.at[pl.ds(base + s * l, l)], ssems[p]
            )

        def refill_and_gather(s, p):
            idx_desc(s, p).start()
            buf = bufs[p]

            @pl.loop(0, l, unroll=2)
            def _row(r):
                for j in range(d // _LANES):
                    c = pl.ds(j * _LANES, _LANES)
                    buf[r, c] = pe_v[r, c]

            idx_desc(s, p).wait()
            for ci in range(nc):
                gather_desc(p, ci, add=True)

        def drain(s, p):
            for ci in range(nc):
                gather_desc(p, ci).wait()
            store_desc(s, p).start()

        @pl.loop(0, seq_per_w // _DEPTH)
        def _grp(g):
            for p in range(_DEPTH):
                s = g * _DEPTH + p

                @pl.when(g > 0)
                def _():
                    store_desc(s - _DEPTH, p).wait()

                refill_and_gather(s, p)

                q = (p - 1) % _DEPTH
                if p >= 1:
                    drain(s - 1, q)
                else:

                    @pl.when(g > 0)
                    def _():
                        drain(s - 1, q)

        last = seq_per_w - 1
        drain(last, _DEPTH - 1)
        for p in range(_DEPTH):
            store_desc(seq_per_w - _DEPTH + p, p).wait()

    return run(x_flat, pe, w)


def kernel(x, W):
    b, l = x.shape
    d = W.shape[1]
    out = _embed_fixed(x.reshape(b * l), W, b, l)
    return out.reshape(b, l, d)


# speedup vs baseline: 2.1007x; 2.1007x over previous
import functools

import jax
import jax.numpy as jnp
import numpy as np
from jax import lax
from jax.experimental import pallas as pl
from jax.experimental.pallas import tpu as pltpu
from jax.experimental.pallas import tpu_sc as plsc

_EMBED = 128
_LANES = 16
_NUM_WORKERS = 32
_DEPTH = 4


def _make_pe(maxlen: int, d: int) -> np.ndarray:
    pe = np.zeros((maxlen, d), dtype=np.float32)
    position = np.arange(0, maxlen)[:, np.newaxis]
    div_term = np.exp(np.arange(0, d, 2) * -(np.log(10000.0) / d))
    pe[:, 0::2] = np.sin(position * div_term)
    pe[:, 1::2] = np.cos(position * div_term)
    return pe


@functools.partial(jax.jit, static_argnums=(2, 3))
def _embed_fixed(x_flat, w, b, l):
    d = w.shape[1]
    n = b * l
    rows_per_w = n // _NUM_WORKERS
    seq_per_w = b // _NUM_WORKERS
    chunks = [(0, 128), (128, l - 128)] if l > 128 else [(0, l)]
    nc = len(chunks)
    pe = jnp.asarray(_make_pe(l, d))
    assert seq_per_w % _DEPTH == 0

    mesh = plsc.VectorSubcoreMesh(core_axis_name="c", subcore_axis_name="s")

    @functools.partial(
        pl.kernel,
        out_type=jax.ShapeDtypeStruct((n, d), jnp.float32),
        mesh=mesh,
        scratch_types=[
            pltpu.VMEM((l, d), jnp.float32),
            [pltpu.VMEM((l,), jnp.int32) for _ in range(_DEPTH)],
            [pltpu.VMEM((l, d), jnp.float32) for _ in range(_DEPTH)],
            [pltpu.SemaphoreType.DMA for _ in range(_DEPTH)],
            [[pltpu.SemaphoreType.DMA for _ in chunks] for _ in range(_DEPTH)],
            [pltpu.SemaphoreType.DMA for _ in range(_DEPTH)],
        ],
    )
    def run(x_hbm, pe_hbm, w_hbm, out_hbm,
            pe_v, idxs, bufs, isems, gsems, ssems):
        wid = lax.axis_index("s") * 2 + lax.axis_index("c")
        base = wid * rows_per_w
        pltpu.sync_copy(pe_hbm, pe_v)

        def idx_desc(s, p):
            return pltpu.make_async_copy(
                x_hbm.at[pl.ds(base + s * l, l)], idxs[p], isems[p]
            )

        def gather_desc(p, ci, add=False):
            off, sz = chunks[ci]
            if add:
                pltpu.async_copy(
                    w_hbm.at[idxs[p].at[pl.ds(off, sz)]],
                    bufs[p].at[pl.ds(off, sz)],
                    gsems[p][ci],
                    add=True,
                )
                return None
            return pltpu.make_async_copy(
                w_hbm.at[idxs[p].at[pl.ds(off, sz)]],
                bufs[p].at[pl.ds(off, sz)],
                gsems[p][ci],
            )

        def store_desc(s, p):
            return pltpu.make_async_copy(
                bufs[p], out_hbm.at[pl.ds(base + s * l, l)], ssems[p]
            )

        def refill_and_gather(s, p):
            buf = bufs[p]

            @pl.loop(0, l)
            def _row(r):
                for j in range(d // _LANES):
                    c = pl.ds(j * _LANES, _LANES)
                    buf[r, c] = pe_v[r, c]

            idx_desc(s, p).wait()
            for ci in range(nc):
                gather_desc(p, ci, add=True)

        def drain(s, p):
            for ci in range(nc):
                gather_desc(p, ci).wait()
            store_desc(s, p).start()

        @pl.loop(0, seq_per_w // _DEPTH)
        def _grp(g):
            for p in range(_DEPTH):
                s = g * _DEPTH + p

                idx_desc(s, p).start()

                @pl.when(g > 0)
                def _():
                    store_desc(s - _DEPTH, p).wait()

                refill_and_gather(s, p)

                q = (p - 1) % _DEPTH
                if p >= 1:
                    drain(s - 1, q)
                else:

                    @pl.when(g > 0)
                    def _():
                        drain(s - 1, q)

        last = seq_per_w - 1
        drain(last, _DEPTH - 1)
        for p in range(_DEPTH):
            store_desc(seq_per_w - _DEPTH + p, p).wait()

    return run(x_flat, pe, w)


def kernel(x, W):
    b, l = x.shape
    d = W.shape[1]
    out = _embed_fixed(x.reshape(b * l), W, b, l)
    return out.reshape(b, l, d)
